# Initial kernel scaffold; baseline (speedup 1.0000x reference)
import jax
import jax.numpy as jnp
from jax.experimental import pallas as pl


def kernel(x, edge_index, edge_attr, Wf1, bf1, Ws1, bs1, Wf2, bf2, Ws2, bs2, Wmu, bmu, Wls, bls):
    raise NotImplementedError("write your pallas kernel here")


# speedup vs baseline: 1.3491x; 1.3491x over previous
import functools

import jax
import jax.numpy as jnp
from jax.experimental import pallas as pl

N_BLK = 1000
E_BLK = 2000
D = 256
DE = 16


def _mm_body(h_ref, w_ref, o_ref):
    o_ref[...] = jnp.dot(h_ref[...], w_ref[...],
                         preferred_element_type=jnp.float32)


def _mm(h, w):
    n, k = h.shape
    m = w.shape[1]
    return pl.pallas_call(
        _mm_body,
        grid=(n // N_BLK,),
        in_specs=[
            pl.BlockSpec((N_BLK, k), lambda i: (i, 0)),
            pl.BlockSpec((k, m), lambda i: (0, 0)),
        ],
        out_specs=pl.BlockSpec((N_BLK, m), lambda i: (i, 0)),
        out_shape=jax.ShapeDtypeStruct((n, m), jnp.float32),
    )(h, w)


def _gate_body(args_ref, ea_ref, we_ref, b_ref, o_ref):
    q = jnp.dot(ea_ref[...], we_ref[...], preferred_element_type=jnp.float32)
    z = args_ref[...] + q + b_ref[...]
    zf = z[:, :D]
    zs = z[:, D:]
    sig = jax.nn.sigmoid(zf)
    sp = jnp.maximum(zs, 0.0) + jnp.log1p(jnp.exp(-jnp.abs(zs)))
    o_ref[...] = sig * sp


def _gate(args, ea, we, bcat):
    e = args.shape[0]
    return pl.pallas_call(
        _gate_body,
        grid=(e // E_BLK,),
        in_specs=[
            pl.BlockSpec((E_BLK, 2 * D), lambda i: (i, 0)),
            pl.BlockSpec((E_BLK, DE), lambda i: (i, 0)),
            pl.BlockSpec((DE, 2 * D), lambda i: (0, 0)),
            pl.BlockSpec((1, 2 * D), lambda i: (0, 0)),
        ],
        out_specs=pl.BlockSpec((E_BLK, D), lambda i: (i, 0)),
        out_shape=jax.ShapeDtypeStruct((e, D), jnp.float32),
    )(args, ea, we, bcat)


def _residual_body(agg_ref, h_ref, o_ref):
    o_ref[...] = jnp.maximum(agg_ref[...] + h_ref[...], 0.0)


def _residual_relu(agg, h):
    n = h.shape[0]
    return pl.pallas_call(
        _residual_body,
        grid=(n // N_BLK,),
        in_specs=[
            pl.BlockSpec((N_BLK, D), lambda i: (i, 0)),
            pl.BlockSpec((N_BLK, D), lambda i: (i, 0)),
        ],
        out_specs=pl.BlockSpec((N_BLK, D), lambda i: (i, 0)),
        out_shape=jax.ShapeDtypeStruct((n, D), jnp.float32),
    )(agg, h)


def _u_body(g_ref, deg_ref, u_ref):
    dinv = jax.lax.rsqrt(deg_ref[...])
    u_ref[...] = dinv * g_ref[...]


def _u_kernel(g, deg):
    n, m = g.shape
    return pl.pallas_call(
        _u_body,
        grid=(n // N_BLK,),
        in_specs=[
            pl.BlockSpec((N_BLK, m), lambda i: (i, 0)),
            pl.BlockSpec((N_BLK, 1), lambda i: (i, 0)),
        ],
        out_specs=pl.BlockSpec((N_BLK, m), lambda i: (i, 0)),
        out_shape=jax.ShapeDtypeStruct((n, m), jnp.float32),
    )(g, deg)


def _final_body(acc_ref, u_ref, deg_ref, bmu_ref, bls_ref, mu_ref, ls_ref):
    dinv = jax.lax.rsqrt(deg_ref[...])
    o = dinv * (acc_ref[...] + u_ref[...])
    mu_ref[...] = o[:, :128] + bmu_ref[...]
    ls_ref[...] = o[:, 128:] + bls_ref[...]


def _final(acc, u, deg, bmu, bls):
    n = acc.shape[0]
    return pl.pallas_call(
        _final_body,
        grid=(n // N_BLK,),
        in_specs=[
            pl.BlockSpec((N_BLK, 2 * 128), lambda i: (i, 0)),
            pl.BlockSpec((N_BLK, 2 * 128), lambda i: (i, 0)),
            pl.BlockSpec((N_BLK, 1), lambda i: (i, 0)),
            pl.BlockSpec((1, 128), lambda i: (0, 0)),
            pl.BlockSpec((1, 128), lambda i: (0, 0)),
        ],
        out_specs=[
            pl.BlockSpec((N_BLK, 128), lambda i: (i, 0)),
            pl.BlockSpec((N_BLK, 128), lambda i: (i, 0)),
        ],
        out_shape=[
            jax.ShapeDtypeStruct((n, 128), jnp.float32),
            jax.ShapeDtypeStruct((n, 128), jnp.float32),
        ],
    )(acc, u, deg, bmu, bls)


def _cg_layer(h, src, dst, ea, Wn, We, bcat):
    t = _mm(h, Wn)
    a = t[:, :2 * D]
    b = t[:, 2 * D:]
    args = a[dst] + b[src]
    m = _gate(args, ea, We, bcat)
    agg = jnp.zeros_like(h).at[dst].add(m)
    return _residual_relu(agg, h)


def kernel(x, edge_index, edge_attr, Wf1, bf1, Ws1, bs1, Wf2, bf2, Ws2, bs2,
           Wmu, bmu, Wls, bls):
    src = edge_index[0]
    dst = edge_index[1]
    n = x.shape[0]

    def pack(Wf, Ws):
        wn = jnp.concatenate([
            jnp.concatenate([Wf[:D], Ws[:D]], axis=1),
            jnp.concatenate([Wf[D:2 * D], Ws[D:2 * D]], axis=1)
        ], axis=1)
        we = jnp.concatenate([Wf[2 * D:], Ws[2 * D:]], axis=1)
        return wn, we

    Wn1, We1 = pack(Wf1, Ws1)
    Wn2, We2 = pack(Wf2, Ws2)
    b1 = jnp.concatenate([bf1, bs1])[None, :]
    b2 = jnp.concatenate([bf2, bs2])[None, :]
    Wg = jnp.concatenate([Wmu, Wls], axis=1)

    deg = (jnp.zeros((n,), jnp.float32).at[dst].add(1.0) + 1.0)[:, None]

    h1 = _cg_layer(x, src, dst, edge_attr, Wn1, We1, b1)
    h2 = _cg_layer(h1, src, dst, edge_attr, Wn2, We2, b2)

    g = _mm(h2, Wg)
    u = _u_kernel(g, deg)
    acc = jnp.zeros((n, 2 * 128), jnp.float32).at[dst].add(u[src])
    mu, ls = _final(acc, u, deg, bmu[None, :], bls[None, :])
    return (mu, ls)


# speedup vs baseline: 2.8117x; 2.0841x over previous
import functools

import jax
import jax.numpy as jnp
from jax import lax
from jax.experimental import pallas as pl
from jax.experimental.pallas import tpu as pltpu
from jax.experimental.pallas import tpu_sc as plsc

N_BLK = 1000
E_BLK = 2000
D = 256
DE = 16
NC = 2
NS = 16


def _mesh():
    return plsc.VectorSubcoreMesh(core_axis_name="c", subcore_axis_name="s")


def _mm2_body(h_ref, w_ref, o1_ref, o2_ref):
    t = jnp.dot(h_ref[...], w_ref[...], preferred_element_type=jnp.float32)
    o1_ref[...] = t[:, :2 * D]
    o2_ref[...] = t[:, 2 * D:]


def _mm2(h, w):
    n, k = h.shape
    return pl.pallas_call(
        _mm2_body,
        grid=(n // N_BLK,),
        in_specs=[
            pl.BlockSpec((N_BLK, k), lambda i: (i, 0)),
            pl.BlockSpec((k, 4 * D), lambda i: (0, 0)),
        ],
        out_specs=[
            pl.BlockSpec((N_BLK, 2 * D), lambda i: (i, 0)),
            pl.BlockSpec((N_BLK, 2 * D), lambda i: (i, 0)),
        ],
        out_shape=[
            jax.ShapeDtypeStruct((n, 2 * D), jnp.float32),
            jax.ShapeDtypeStruct((n, 2 * D), jnp.float32),
        ],
    )(h, w)


def _mm_body(h_ref, w_ref, o_ref):
    o_ref[...] = jnp.dot(h_ref[...], w_ref[...],
                         preferred_element_type=jnp.float32)


def _mm(h, w):
    n, k = h.shape
    m = w.shape[1]
    return pl.pallas_call(
        _mm_body,
        grid=(n // N_BLK,),
        in_specs=[
            pl.BlockSpec((N_BLK, k), lambda i: (i, 0)),
            pl.BlockSpec((k, m), lambda i: (0, 0)),
        ],
        out_specs=pl.BlockSpec((N_BLK, m), lambda i: (i, 0)),
        out_shape=jax.ShapeDtypeStruct((n, m), jnp.float32),
    )(h, w)


def _gate_body(args_ref, ea_ref, we_ref, b_ref, o_ref):
    q = jnp.dot(ea_ref[...], we_ref[...], preferred_element_type=jnp.float32)
    z = args_ref[...] + q + b_ref[...]
    zf = z[:, :D]
    zs = z[:, D:]
    sig = jax.nn.sigmoid(zf)
    sp = jnp.maximum(zs, 0.0) + jnp.log1p(jnp.exp(-jnp.abs(zs)))
    m = sig * sp
    o_ref[0] = m[:, :128]
    o_ref[1] = m[:, 128:]


def _gate(args, ea, we, bcat):
    e = args.shape[0]
    return pl.pallas_call(
        _gate_body,
        grid=(e // E_BLK,),
        in_specs=[
            pl.BlockSpec((E_BLK, 2 * D), lambda i: (i, 0)),
            pl.BlockSpec((E_BLK, DE), lambda i: (i, 0)),
            pl.BlockSpec((DE, 2 * D), lambda i: (0, 0)),
            pl.BlockSpec((1, 2 * D), lambda i: (0, 0)),
        ],
        out_specs=pl.BlockSpec((2, E_BLK, 128), lambda i: (0, i, 0)),
        out_shape=jax.ShapeDtypeStruct((2, e, 128), jnp.float32),
    )(args, ea, we, bcat)


def _residual_body(a0_ref, a1_ref, h_ref, o_ref):
    agg = jnp.concatenate([a0_ref[...], a1_ref[...]], axis=1)
    o_ref[...] = jnp.maximum(agg + h_ref[...], 0.0)


def _residual_relu(agg2, h):
    n = h.shape[0]
    nb = n // N_BLK
    return pl.pallas_call(
        _residual_body,
        grid=(nb,),
        in_specs=[
            pl.BlockSpec((N_BLK, 128), lambda i: (i, 0)),
            pl.BlockSpec((N_BLK, 128), lambda i, _nb=nb: (i + _nb, 0)),
            pl.BlockSpec((N_BLK, D), lambda i: (i, 0)),
        ],
        out_specs=pl.BlockSpec((N_BLK, D), lambda i: (i, 0)),
        out_shape=jax.ShapeDtypeStruct((n, D), jnp.float32),
    )(agg2, agg2, h)


def _u_body(g_ref, d0_ref, d1_ref, u_ref):
    deg = d0_ref[:, :1] + d1_ref[:, :1] + 1.0
    u_ref[...] = jax.lax.rsqrt(deg) * g_ref[...]


def _u_kernel(g, degp):
    n = g.shape[0]
    nb = n // N_BLK
    return pl.pallas_call(
        _u_body,
        grid=(2, nb),
        in_specs=[
            pl.BlockSpec((N_BLK, 128), lambda c, i: (i, c)),
            pl.BlockSpec((N_BLK, 128), lambda c, i: (i, 0)),
            pl.BlockSpec((N_BLK, 128), lambda c, i, _nb=nb: (i + _nb, 0)),
        ],
        out_specs=pl.BlockSpec((N_BLK, 128), lambda c, i, _nb=nb: (c * _nb + i, 0)),
        out_shape=jax.ShapeDtypeStruct((2 * n, 128), jnp.float32),
    )(g, degp, degp)


def _final_body(a0_ref, a1_ref, u0_ref, u1_ref, d0_ref, d1_ref,
                bmu_ref, bls_ref, mu_ref, ls_ref):
    dinv = jax.lax.rsqrt(d0_ref[:, :1] + d1_ref[:, :1] + 1.0)
    mu_ref[...] = dinv * (a0_ref[...] + u0_ref[...]) + bmu_ref[...]
    ls_ref[...] = dinv * (a1_ref[...] + u1_ref[...]) + bls_ref[...]


def _final(acc2, u2, degp, bmu, bls):
    n = acc2.shape[0] // 2
    nb = n // N_BLK
    return pl.pallas_call(
        _final_body,
        grid=(nb,),
        in_specs=[
            pl.BlockSpec((N_BLK, 128), lambda i: (i, 0)),
            pl.BlockSpec((N_BLK, 128), lambda i, _nb=nb: (i + _nb, 0)),
            pl.BlockSpec((N_BLK, 128), lambda i: (i, 0)),
            pl.BlockSpec((N_BLK, 128), lambda i, _nb=nb: (i + _nb, 0)),
            pl.BlockSpec((N_BLK, 128), lambda i: (i, 0)),
            pl.BlockSpec((N_BLK, 128), lambda i, _nb=nb: (i + _nb, 0)),
            pl.BlockSpec((1, 128), lambda i: (0, 0)),
            pl.BlockSpec((1, 128), lambda i: (0, 0)),
        ],
        out_specs=[
            pl.BlockSpec((N_BLK, 128), lambda i: (i, 0)),
            pl.BlockSpec((N_BLK, 128), lambda i: (i, 0)),
        ],
        out_shape=[
            jax.ShapeDtypeStruct((n, 128), jnp.float32),
            jax.ShapeDtypeStruct((n, 128), jnp.float32),
        ],
    )(acc2, acc2, u2, u2, degp, degp, bmu, bls)


def _sc_args(tdst, tsrc, src_i, dst_i):
    e = src_i.shape[0]
    n = tdst.shape[0]
    c = 80
    per_tile = e // (NC * NS)
    nfull = per_tile // c
    last_off = per_tile - c

    @functools.partial(
        pl.kernel,
        out_type=jax.ShapeDtypeStruct((e, 2 * D), jnp.float32),
        mesh=_mesh(),
        scratch_types=[
            pltpu.VMEM((c,), jnp.int32),
            pltpu.VMEM((c,), jnp.int32),
            pltpu.VMEM((c, 2 * D), jnp.float32),
            pltpu.VMEM((c, 2 * D), jnp.float32),
            pltpu.SemaphoreType.DMA,
            pltpu.SemaphoreType.DMA,
        ],
    )
    def k(tdst_h, tsrc_h, src_h, dst_h, out_h, idxd, idxs, bufa, bufb,
          sema, semb):
        wid = lax.axis_index("c") * NS + lax.axis_index("s")
        base = wid * per_tile

        def chunk(eb):
            eb = pl.multiple_of(eb, 8)
            pltpu.sync_copy(dst_h.at[pl.ds(eb, c)], idxd)
            pltpu.sync_copy(src_h.at[pl.ds(eb, c)], idxs)
            ca = pltpu.async_copy(tdst_h.at[idxd], bufa, sema)
            cb = pltpu.async_copy(tsrc_h.at[idxs], bufb, semb)
            ca.wait()
            cb.wait()

            def addrow(r, carry):
                for g in range(2 * D // 16):
                    plsc.addupdate(bufa.at[r, pl.ds(g * 16, 16)],
                                   bufb[r, pl.ds(g * 16, 16)])
                return carry

            lax.fori_loop(0, c, addrow, 0)
            pltpu.sync_copy(bufa, out_h.at[pl.ds(eb, c)])

        def body(j, carry):
            chunk(base + j * c)
            return carry

        lax.fori_loop(0, nfull, body, 0)
        chunk(base + last_off)

    return k(tdst, tsrc, src_i, dst_i)


def _row_chunks(n, c):
    assert n % c == 0
    return n // c


def _sc_scatter(m2, dst_i, n):
    e = dst_i.shape[0]
    c = 80
    per_tile = e // NS
    nchunks = per_tile // c
    nrow_chunks = _row_chunks(n, c)

    @functools.partial(
        pl.kernel,
        out_type=jax.ShapeDtypeStruct((2 * n, 128), jnp.float32),
        mesh=_mesh(),
        scratch_types=[
            pltpu.VMEM((c,), jnp.int32),
            pltpu.VMEM((c, 128), jnp.float32),
            pltpu.VMEM_SHARED((n, 128), jnp.float32),
        ],
    )
    def k(m_h, dst_h, out_h, idx, buf, acc):
        cid = lax.axis_index("c")
        sid = lax.axis_index("s")
        z16 = jnp.zeros((16,), jnp.float32)

        def zrow(r, carry):
            for g in range(8):
                buf[r, pl.ds(g * 16, 16)] = z16
            return carry

        lax.fori_loop(0, c, zrow, 0)
        for jj in range((nrow_chunks + NS - 1) // NS):
            ck = jj * NS + sid

            @pl.when(ck < nrow_chunks)
            def _(ck=ck):
                off = pl.multiple_of(ck * c, 8)
                pltpu.sync_copy(buf, acc.at[pl.ds(off, c)])

        plsc.subcore_barrier()

        base = sid * per_tile

        def body(j, carry):
            eb = pl.multiple_of(base + j * c, 8)
            pltpu.sync_copy(dst_h.at[pl.ds(eb, c)], idx)
            pltpu.sync_copy(m_h.at[pl.ds(pl.multiple_of(cid * e + eb, 8), c)],
                            buf)
            pltpu.sync_copy(buf, acc.at[idx], add=True)
            return carry

        lax.fori_loop(0, nchunks, body, 0)
        plsc.subcore_barrier()
        for jj in range((nrow_chunks + NS - 1) // NS):
            ck = jj * NS + sid

            @pl.when(ck < nrow_chunks)
            def _(ck=ck):
                off = pl.multiple_of(ck * c, 8)
                oo = pl.multiple_of(cid * n + ck * c, 8)
                pltpu.sync_copy(acc.at[pl.ds(off, c)], out_h.at[pl.ds(oo, c)])

    return k(m2, dst_i)


def _sc_deg(dst_i, n):
    e = dst_i.shape[0]
    c = 40
    per_tile = e // (NC * NS)
    nchunks = per_tile // c
    zc = 80

    @functools.partial(
        pl.kernel,
        out_type=jax.ShapeDtypeStruct((2 * n, 128), jnp.float32),
        mesh=_mesh(),
        scratch_types=[
            pltpu.VMEM((c,), jnp.int32),
            pltpu.VMEM((zc, 128), jnp.float32),
            pltpu.VMEM_SHARED((n, 128), jnp.float32),
        ],
    )
    def k(dst_h, out_h, idx, buf, dacc):
        cid = lax.axis_index("c")
        sid = lax.axis_index("s")
        wid = cid * NS + sid
        z16 = jnp.zeros((16,), jnp.float32)
        o16 = jnp.ones((16,), jnp.float32)
        nrow_chunks = n // zc

        def zrow(r, carry):
            for g in range(8):
                buf[r, pl.ds(g * 16, 16)] = z16
            return carry

        lax.fori_loop(0, zc, zrow, 0)
        for jj in range((nrow_chunks + NS - 1) // NS):
            ck = jj * NS + sid

            @pl.when(ck < nrow_chunks)
            def _(ck=ck):
                off = pl.multiple_of(ck * zc, 8)
                pltpu.sync_copy(buf, dacc.at[pl.ds(off, zc)])

        def orow(r, carry):
            for g in range(8):
                buf[r, pl.ds(g * 16, 16)] = o16
            return carry

        lax.fori_loop(0, c, orow, 0)
        plsc.subcore_barrier()

        base = wid * per_tile

        def body(j, carry):
            eb = pl.multiple_of(base + j * c, 8)
            pltpu.sync_copy(dst_h.at[pl.ds(eb, c)], idx)
            pltpu.sync_copy(buf.at[pl.ds(0, c)], dacc.at[idx], add=True)
            return carry

        lax.fori_loop(0, nchunks, body, 0)
        plsc.subcore_barrier()
        for jj in range((nrow_chunks + NS - 1) // NS):
            ck = jj * NS + sid

            @pl.when(ck < nrow_chunks)
            def _(ck=ck):
                off = pl.multiple_of(ck * zc, 8)
                oo = pl.multiple_of(cid * n + ck * zc, 8)
                pltpu.sync_copy(dacc.at[pl.ds(off, zc)],
                                out_h.at[pl.ds(oo, zc)])

    return k(dst_i)


def _sc_gcn(u2, src_off, dst_i, n):
    e = dst_i.shape[0]
    c = 80
    per_tile = e // NS
    nchunks = per_tile // c
    rpt = n // NS

    @functools.partial(
        pl.kernel,
        out_type=jax.ShapeDtypeStruct((2 * n, 128), jnp.float32),
        mesh=_mesh(),
        scratch_types=[
            pltpu.VMEM((c,), jnp.int32),
            pltpu.VMEM((c,), jnp.int32),
            pltpu.VMEM((c, 128), jnp.float32),
            pltpu.VMEM_SHARED((n, 128), jnp.float32),
            pltpu.SemaphoreType.DMA,
        ],
    )
    def k(u_h, src_h, dst_h, out_h, idxs, idxd, buf, acc, sem):
        cid = lax.axis_index("c")
        sid = lax.axis_index("s")
        z16 = jnp.zeros((16,), jnp.float32)
        nrow_chunks = n // c

        def zrow(r, carry):
            for g in range(8):
                buf[r, pl.ds(g * 16, 16)] = z16
            return carry

        lax.fori_loop(0, c, zrow, 0)
        for jj in range((nrow_chunks + NS - 1) // NS):
            ck = jj * NS + sid

            @pl.when(ck < nrow_chunks)
            def _(ck=ck):
                off = pl.multiple_of(ck * c, 8)
                pltpu.sync_copy(buf, acc.at[pl.ds(off, c)])

        plsc.subcore_barrier()

        base = sid * per_tile

        def body(j, carry):
            eb = pl.multiple_of(base + j * c, 8)
            pltpu.sync_copy(src_h.at[pl.ds(pl.multiple_of(cid * e + eb, 8), c)],
                            idxs)
            pltpu.sync_copy(dst_h.at[pl.ds(eb, c)], idxd)
            pltpu.async_copy(u_h.at[idxs], buf, sem).wait()
            pltpu.sync_copy(buf, acc.at[idxd], add=True)
            return carry

        lax.fori_loop(0, nchunks, body, 0)
        plsc.subcore_barrier()
        for jj in range((nrow_chunks + NS - 1) // NS):
            ck = jj * NS + sid

            @pl.when(ck < nrow_chunks)
            def _(ck=ck):
                off = pl.multiple_of(ck * c, 8)
                oo = pl.multiple_of(cid * n + ck * c, 8)
                pltpu.sync_copy(acc.at[pl.ds(off, c)], out_h.at[pl.ds(oo, c)])

    return k(u2, src_off, dst_i)


def _cg_layer(h, src_i, dst_i, ea, wn, we, bcat):
    tdst, tsrc = _mm2(h, wn)
    args = _sc_args(tdst, tsrc, src_i, dst_i)
    m = _gate(args, ea, we, bcat)
    m2 = jnp.reshape(m, (-1, 128))
    agg2 = _sc_scatter(m2, dst_i, h.shape[0])
    return _residual_relu(agg2, h)


def kernel(x, edge_index, edge_attr, Wf1, bf1, Ws1, bs1, Wf2, bf2, Ws2, bs2,
           Wmu, bmu, Wls, bls):
    src_i = edge_index[0]
    dst_i = edge_index[1]
    n = x.shape[0]

    def pack(wf, ws):
        wn = jnp.concatenate([
            jnp.concatenate([wf[:D], ws[:D]], axis=1),
            jnp.concatenate([wf[D:2 * D], ws[D:2 * D]], axis=1),
        ], axis=1)
        we = jnp.concatenate([wf[2 * D:], ws[2 * D:]], axis=1)
        return wn, we

    wn1, we1 = pack(Wf1, Ws1)
    wn2, we2 = pack(Wf2, Ws2)
    b1 = jnp.concatenate([bf1, bs1])[None, :]
    b2 = jnp.concatenate([bf2, bs2])[None, :]
    wg = jnp.concatenate([Wmu, Wls], axis=1)
    src_off = jnp.concatenate([src_i, src_i + n])

    degp = _sc_deg(dst_i, n)

    h1 = _cg_layer(x, src_i, dst_i, edge_attr, wn1, we1, b1)
    h2 = _cg_layer(h1, src_i, dst_i, edge_attr, wn2, we2, b2)

    g = _mm(h2, wg)
    u2 = _u_kernel(g, degp)
    acc2 = _sc_gcn(u2, src_off, dst_i, n)
    mu, ls = _final(acc2, u2, degp, bmu[None, :], bls[None, :])
    return (mu, ls)


# speedup vs baseline: 3.8785x; 1.3794x over previous
import functools

import jax
import jax.numpy as jnp
from jax import lax
from jax.experimental import pallas as pl
from jax.experimental.pallas import tpu as pltpu
from jax.experimental.pallas import tpu_sc as plsc

N_BLK = 1000
E_BLK = 2000
D = 256
DE = 16
NC = 2
NS = 16


def _mesh():
    return plsc.VectorSubcoreMesh(core_axis_name="c", subcore_axis_name="s")


def _mm2_body(h_ref, w_ref, o1_ref, o2_ref):
    t = jnp.dot(h_ref[...], w_ref[...], preferred_element_type=jnp.float32)
    o1_ref[...] = t[:, :2 * D]
    o2_ref[...] = t[:, 2 * D:]


def _mm2(h, w):
    n, k = h.shape
    return pl.pallas_call(
        _mm2_body,
        grid=(n // N_BLK,),
        in_specs=[
            pl.BlockSpec((N_BLK, k), lambda i: (i, 0)),
            pl.BlockSpec((k, 4 * D), lambda i: (0, 0)),
        ],
        out_specs=[
            pl.BlockSpec((N_BLK, 2 * D), lambda i: (i, 0)),
            pl.BlockSpec((N_BLK, 2 * D), lambda i: (i, 0)),
        ],
        out_shape=[
            jax.ShapeDtypeStruct((n, 2 * D), jnp.float32),
            jax.ShapeDtypeStruct((n, 2 * D), jnp.float32),
        ],
    )(h, w)


def _mm_body(h_ref, w_ref, o_ref):
    o_ref[...] = jnp.dot(h_ref[...], w_ref[...],
                         preferred_element_type=jnp.float32)


def _mm(h, w):
    n, k = h.shape
    m = w.shape[1]
    return pl.pallas_call(
        _mm_body,
        grid=(n // N_BLK,),
        in_specs=[
            pl.BlockSpec((N_BLK, k), lambda i: (i, 0)),
            pl.BlockSpec((k, m), lambda i: (0, 0)),
        ],
        out_specs=pl.BlockSpec((N_BLK, m), lambda i: (i, 0)),
        out_shape=jax.ShapeDtypeStruct((n, m), jnp.float32),
    )(h, w)


def _gate_body(args_ref, ea_ref, we_ref, b_ref, o_ref):
    q = jnp.dot(ea_ref[...], we_ref[...], preferred_element_type=jnp.float32)
    z = args_ref[...] + q + b_ref[...]
    zf = z[:, :D]
    zs = z[:, D:]
    sig = jax.nn.sigmoid(zf)
    sp = jnp.maximum(zs, 0.0) + jnp.log1p(jnp.exp(-jnp.abs(zs)))
    m = sig * sp
    o_ref[0] = m[:, :128]
    o_ref[1] = m[:, 128:]


def _gate(args, ea, we, bcat):
    e = args.shape[0]
    return pl.pallas_call(
        _gate_body,
        grid=(e // E_BLK,),
        in_specs=[
            pl.BlockSpec((E_BLK, 2 * D), lambda i: (i, 0)),
            pl.BlockSpec((E_BLK, DE), lambda i: (i, 0)),
            pl.BlockSpec((DE, 2 * D), lambda i: (0, 0)),
            pl.BlockSpec((1, 2 * D), lambda i: (0, 0)),
        ],
        out_specs=pl.BlockSpec((2, E_BLK, 128), lambda i: (0, i, 0)),
        out_shape=jax.ShapeDtypeStruct((2, e, 128), jnp.float32),
    )(args, ea, we, bcat)


def _residual_body(a0_ref, a1_ref, h_ref, o_ref):
    agg = jnp.concatenate([a0_ref[...], a1_ref[...]], axis=1)
    o_ref[...] = jnp.maximum(agg + h_ref[...], 0.0)


def _residual_relu(agg2, h):
    n = h.shape[0]
    nb = n // N_BLK
    return pl.pallas_call(
        _residual_body,
        grid=(nb,),
        in_specs=[
            pl.BlockSpec((N_BLK, 128), lambda i: (i, 0)),
            pl.BlockSpec((N_BLK, 128), lambda i, _nb=nb: (i + _nb, 0)),
            pl.BlockSpec((N_BLK, D), lambda i: (i, 0)),
        ],
        out_specs=pl.BlockSpec((N_BLK, D), lambda i: (i, 0)),
        out_shape=jax.ShapeDtypeStruct((n, D), jnp.float32),
    )(agg2, agg2, h)


def _u_body(g_ref, d0_ref, d1_ref, u_ref):
    deg = d0_ref[:, :1] + d1_ref[:, :1] + 1.0
    u_ref[...] = jax.lax.rsqrt(deg) * g_ref[...]


def _u_kernel(g, degp):
    n = g.shape[0]
    nb = n // N_BLK
    return pl.pallas_call(
        _u_body,
        grid=(2, nb),
        in_specs=[
            pl.BlockSpec((N_BLK, 128), lambda c, i: (i, c)),
            pl.BlockSpec((N_BLK, 128), lambda c, i: (i, 0)),
            pl.BlockSpec((N_BLK, 128), lambda c, i, _nb=nb: (i + _nb, 0)),
        ],
        out_specs=pl.BlockSpec((N_BLK, 128), lambda c, i, _nb=nb: (c * _nb + i, 0)),
        out_shape=jax.ShapeDtypeStruct((2 * n, 128), jnp.float32),
    )(g, degp, degp)


def _final_body(a0_ref, a1_ref, u0_ref, u1_ref, d0_ref, d1_ref,
                bmu_ref, bls_ref, mu_ref, ls_ref):
    dinv = jax.lax.rsqrt(d0_ref[:, :1] + d1_ref[:, :1] + 1.0)
    mu_ref[...] = dinv * (a0_ref[...] + u0_ref[...]) + bmu_ref[...]
    ls_ref[...] = dinv * (a1_ref[...] + u1_ref[...]) + bls_ref[...]


def _final(acc2, u2, degp, bmu, bls):
    n = acc2.shape[0] // 2
    nb = n // N_BLK
    return pl.pallas_call(
        _final_body,
        grid=(nb,),
        in_specs=[
            pl.BlockSpec((N_BLK, 128), lambda i: (i, 0)),
            pl.BlockSpec((N_BLK, 128), lambda i, _nb=nb: (i + _nb, 0)),
            pl.BlockSpec((N_BLK, 128), lambda i: (i, 0)),
            pl.BlockSpec((N_BLK, 128), lambda i, _nb=nb: (i + _nb, 0)),
            pl.BlockSpec((N_BLK, 128), lambda i: (i, 0)),
            pl.BlockSpec((N_BLK, 128), lambda i, _nb=nb: (i + _nb, 0)),
            pl.BlockSpec((1, 128), lambda i: (0, 0)),
            pl.BlockSpec((1, 128), lambda i: (0, 0)),
        ],
        out_specs=[
            pl.BlockSpec((N_BLK, 128), lambda i: (i, 0)),
            pl.BlockSpec((N_BLK, 128), lambda i: (i, 0)),
        ],
        out_shape=[
            jax.ShapeDtypeStruct((n, 128), jnp.float32),
            jax.ShapeDtypeStruct((n, 128), jnp.float32),
        ],
    )(acc2, acc2, u2, u2, degp, degp, bmu, bls)


def _sc_args(tdst, tsrc, src_i, dst_i):
    e = src_i.shape[0]
    c = 56
    per_tile = e // (NC * NS)
    nfull = per_tile // c
    nfull -= nfull % 2
    tail1 = per_tile - 2 * c
    tail2 = per_tile - c

    @functools.partial(
        pl.kernel,
        out_type=jax.ShapeDtypeStruct((e, 2 * D), jnp.float32),
        mesh=_mesh(),
        scratch_types=[
            pltpu.VMEM((c,), jnp.int32),
            pltpu.VMEM((c,), jnp.int32),
            pltpu.VMEM((c,), jnp.int32),
            pltpu.VMEM((c,), jnp.int32),
            pltpu.VMEM((c, 2 * D), jnp.float32),
            pltpu.VMEM((c, 2 * D), jnp.float32),
            pltpu.VMEM((c, 2 * D), jnp.float32),
            pltpu.VMEM((c, 2 * D), jnp.float32),
            pltpu.SemaphoreType.DMA,
            pltpu.SemaphoreType.DMA,
            pltpu.SemaphoreType.DMA,
            pltpu.SemaphoreType.DMA,
        ],
    )
    def k(tdst_h, tsrc_h, src_h, dst_h, out_h, idxd0, idxd1, idxs0, idxs1,
          bufa0, bufa1, bufb0, bufb1, sa0, sa1, sb0, sb1):
        wid = lax.axis_index("c") * NS + lax.axis_index("s")
        base = wid * per_tile
        idxd = (idxd0, idxd1)
        idxs = (idxs0, idxs1)
        bufa = (bufa0, bufa1)
        bufb = (bufb0, bufb1)
        sa = (sa0, sa1)
        sb = (sb0, sb1)

        def start(s, eb):
            eb = pl.multiple_of(eb, 8)
            pltpu.sync_copy(dst_h.at[pl.ds(eb, c)], idxd[s])
            pltpu.sync_copy(src_h.at[pl.ds(eb, c)], idxs[s])
            pltpu.async_copy(tdst_h.at[idxd[s]], bufa[s], sa[s])
            pltpu.async_copy(tsrc_h.at[idxs[s]], bufb[s], sb[s])

        def finish(s, eb):
            eb = pl.multiple_of(eb, 8)
            pltpu.make_async_copy(tdst_h.at[idxd[s]], bufa[s], sa[s]).wait()
            pltpu.make_async_copy(tsrc_h.at[idxs[s]], bufb[s], sb[s]).wait()

            def addrow(r, carry):
                for g in range(2 * D // 16):
                    plsc.addupdate(bufa[s].at[r, pl.ds(g * 16, 16)],
                                   bufb[s][r, pl.ds(g * 16, 16)])
                return carry

            lax.fori_loop(0, c, addrow, 0)
            pltpu.sync_copy(bufa[s], out_h.at[pl.ds(eb, c)])

        start(0, base)
        start(1, base + c)

        def body(j2, carry):
            j = 2 * j2
            finish(0, base + j * c)

            @pl.when(j + 2 < nfull)
            def _():
                start(0, base + (j + 2) * c)

            finish(1, base + (j + 1) * c)

            @pl.when(j + 3 < nfull)
            def _():
                start(1, base + (j + 3) * c)

            return carry

        lax.fori_loop(0, nfull // 2, body, 0)
        start(0, base + tail1)
        start(1, base + tail2)
        finish(0, base + tail1)
        finish(1, base + tail2)

    return k(tdst, tsrc, src_i, dst_i)


def _row_chunks(n, c):
    assert n % c == 0
    return n // c


def _sc_scatter(m2, dst_i, n):
    e = dst_i.shape[0]
    c = 80
    per_tile = e // NS
    nchunks = per_tile // c
    nrow_chunks = _row_chunks(n, c)

    @functools.partial(
        pl.kernel,
        out_type=jax.ShapeDtypeStruct((2 * n, 128), jnp.float32),
        mesh=_mesh(),
        scratch_types=[
            pltpu.VMEM((c,), jnp.int32),
            pltpu.VMEM((c,), jnp.int32),
            pltpu.VMEM((c, 128), jnp.float32),
            pltpu.VMEM((c, 128), jnp.float32),
            pltpu.VMEM_SHARED((n, 128), jnp.float32),
            pltpu.SemaphoreType.DMA,
            pltpu.SemaphoreType.DMA,
        ],
    )
    def k(m_h, dst_h, out_h, idx0, idx1, buf0, buf1, acc, sm0, sm1):
        cid = lax.axis_index("c")
        sid = lax.axis_index("s")
        z16 = jnp.zeros((16,), jnp.float32)
        idx = (idx0, idx1)
        buf = (buf0, buf1)
        sm = (sm0, sm1)

        def zrow(r, carry):
            for g in range(8):
                buf0[r, pl.ds(g * 16, 16)] = z16
            return carry

        lax.fori_loop(0, c, zrow, 0)
        for jj in range((nrow_chunks + NS - 1) // NS):
            ck = jj * NS + sid

            @pl.when(ck < nrow_chunks)
            def _(ck=ck):
                off = pl.multiple_of(ck * c, 8)
                pltpu.sync_copy(buf0, acc.at[pl.ds(off, c)])

        plsc.subcore_barrier()

        base = sid * per_tile

        def start(s, eb):
            eb = pl.multiple_of(eb, 8)
            pltpu.sync_copy(dst_h.at[pl.ds(eb, c)], idx[s])
            pltpu.async_copy(
                m_h.at[pl.ds(pl.multiple_of(cid * e + eb, 8), c)],
                buf[s], sm[s])

        def finish(s, eb):
            eb = pl.multiple_of(eb, 8)
            pltpu.make_async_copy(
                m_h.at[pl.ds(pl.multiple_of(cid * e + eb, 8), c)],
                buf[s], sm[s]).wait()
            pltpu.sync_copy(buf[s], acc.at[idx[s]], add=True)

        start(0, base)
        start(1, base + c)

        def body(j2, carry):
            j = 2 * j2
            finish(0, base + j * c)

            @pl.when(j + 2 < nchunks)
            def _():
                start(0, base + (j + 2) * c)

            finish(1, base + (j + 1) * c)

            @pl.when(j + 3 < nchunks)
            def _():
                start(1, base + (j + 3) * c)

            return carry

        lax.fori_loop(0, (nchunks - 1) // 2, body, 0)
        finish((nchunks - 1) % 2, base + (nchunks - 1) * c)
        plsc.subcore_barrier()
        for jj in range((nrow_chunks + NS - 1) // NS):
            ck = jj * NS + sid

            @pl.when(ck < nrow_chunks)
            def _(ck=ck):
                off = pl.multiple_of(ck * c, 8)
                oo = pl.multiple_of(cid * n + ck * c, 8)
                pltpu.sync_copy(acc.at[pl.ds(off, c)], out_h.at[pl.ds(oo, c)])

    return k(m2, dst_i)


def _sc_deg(dst_i, n):
    e = dst_i.shape[0]
    c = 40
    per_tile = e // (NC * NS)
    nchunks = per_tile // c
    zc = 80

    @functools.partial(
        pl.kernel,
        out_type=jax.ShapeDtypeStruct((2 * n, 128), jnp.float32),
        mesh=_mesh(),
        scratch_types=[
            pltpu.VMEM((c,), jnp.int32),
            pltpu.VMEM((zc, 128), jnp.float32),
            pltpu.VMEM_SHARED((n, 128), jnp.float32),
        ],
    )
    def k(dst_h, out_h, idx, buf, dacc):
        cid = lax.axis_index("c")
        sid = lax.axis_index("s")
        wid = cid * NS + sid
        z16 = jnp.zeros((16,), jnp.float32)
        o16 = jnp.ones((16,), jnp.float32)
        nrow_chunks = n // zc

        def zrow(r, carry):
            for g in range(8):
                buf[r, pl.ds(g * 16, 16)] = z16
            return carry

        lax.fori_loop(0, zc, zrow, 0)
        for jj in range((nrow_chunks + NS - 1) // NS):
            ck = jj * NS + sid

            @pl.when(ck < nrow_chunks)
            def _(ck=ck):
                off = pl.multiple_of(ck * zc, 8)
                pltpu.sync_copy(buf, dacc.at[pl.ds(off, zc)])

        def orow(r, carry):
            for g in range(8):
                buf[r, pl.ds(g * 16, 16)] = o16
            return carry

        lax.fori_loop(0, c, orow, 0)
        plsc.subcore_barrier()

        base = wid * per_tile

        def body(j, carry):
            eb = pl.multiple_of(base + j * c, 8)
            pltpu.sync_copy(dst_h.at[pl.ds(eb, c)], idx)
            pltpu.sync_copy(buf.at[pl.ds(0, c)], dacc.at[idx], add=True)
            return carry

        lax.fori_loop(0, nchunks, body, 0)
        plsc.subcore_barrier()
        for jj in range((nrow_chunks + NS - 1) // NS):
            ck = jj * NS + sid

            @pl.when(ck < nrow_chunks)
            def _(ck=ck):
                off = pl.multiple_of(ck * zc, 8)
                oo = pl.multiple_of(cid * n + ck * zc, 8)
                pltpu.sync_copy(dacc.at[pl.ds(off, zc)],
                                out_h.at[pl.ds(oo, zc)])

    return k(dst_i)


def _sc_gcn(u2, src_off, dst_i, n):
    e = dst_i.shape[0]
    c = 80
    per_tile = e // NS
    nchunks = per_tile // c
    rpt = n // NS

    @functools.partial(
        pl.kernel,
        out_type=jax.ShapeDtypeStruct((2 * n, 128), jnp.float32),
        mesh=_mesh(),
        scratch_types=[
            pltpu.VMEM((c,), jnp.int32),
            pltpu.VMEM((c,), jnp.int32),
            pltpu.VMEM((c,), jnp.int32),
            pltpu.VMEM((c,), jnp.int32),
            pltpu.VMEM((c, 128), jnp.float32),
            pltpu.VMEM((c, 128), jnp.float32),
            pltpu.VMEM_SHARED((n, 128), jnp.float32),
            pltpu.SemaphoreType.DMA,
            pltpu.SemaphoreType.DMA,
        ],
    )
    def k(u_h, src_h, dst_h, out_h, idxs0, idxs1, idxd0, idxd1, buf0, buf1,
          acc, sg0, sg1):
        cid = lax.axis_index("c")
        sid = lax.axis_index("s")
        z16 = jnp.zeros((16,), jnp.float32)
        nrow_chunks = n // c
        idxs = (idxs0, idxs1)
        idxd = (idxd0, idxd1)
        buf = (buf0, buf1)
        sg = (sg0, sg1)

        def zrow(r, carry):
            for g in range(8):
                buf0[r, pl.ds(g * 16, 16)] = z16
            return carry

        lax.fori_loop(0, c, zrow, 0)
        for jj in range((nrow_chunks + NS - 1) // NS):
            ck = jj * NS + sid

            @pl.when(ck < nrow_chunks)
            def _(ck=ck):
                off = pl.multiple_of(ck * c, 8)
                pltpu.sync_copy(buf0, acc.at[pl.ds(off, c)])

        plsc.subcore_barrier()

        base = sid * per_tile

        def start(s, eb):
            eb = pl.multiple_of(eb, 8)
            pltpu.sync_copy(src_h.at[pl.ds(pl.multiple_of(cid * e + eb, 8), c)],
                            idxs[s])
            pltpu.sync_copy(dst_h.at[pl.ds(eb, c)], idxd[s])
            pltpu.async_copy(u_h.at[idxs[s]], buf[s], sg[s])

        def finish(s):
            pltpu.make_async_copy(u_h.at[idxs[s]], buf[s], sg[s]).wait()
            pltpu.sync_copy(buf[s], acc.at[idxd[s]], add=True)

        start(0, base)
        start(1, base + c)

        def body(j2, carry):
            j = 2 * j2
            finish(0)

            @pl.when(j + 2 < nchunks)
            def _():
                start(0, base + (j + 2) * c)

            finish(1)

            @pl.when(j + 3 < nchunks)
            def _():
                start(1, base + (j + 3) * c)

            return carry

        lax.fori_loop(0, (nchunks - 1) // 2, body, 0)
        finish((nchunks - 1) % 2)
        plsc.subcore_barrier()
        for jj in range((nrow_chunks + NS - 1) // NS):
            ck = jj * NS + sid

            @pl.when(ck < nrow_chunks)
            def _(ck=ck):
                off = pl.multiple_of(ck * c, 8)
                oo = pl.multiple_of(cid * n + ck * c, 8)
                pltpu.sync_copy(acc.at[pl.ds(off, c)], out_h.at[pl.ds(oo, c)])

    return k(u2, src_off, dst_i)


def _cg_layer(h, src_i, dst_i, ea, wn, we, bcat):
    tdst, tsrc = _mm2(h, wn)
    args = _sc_args(tdst, tsrc, src_i, dst_i)
    m = _gate(args, ea, we, bcat)
    m2 = jnp.reshape(m, (-1, 128))
    agg2 = _sc_scatter(m2, dst_i, h.shape[0])
    return _residual_relu(agg2, h)


def kernel(x, edge_index, edge_attr, Wf1, bf1, Ws1, bs1, Wf2, bf2, Ws2, bs2,
           Wmu, bmu, Wls, bls):
    src_i = edge_index[0]
    dst_i = edge_index[1]
    n = x.shape[0]

    def pack(wf, ws):
        wn = jnp.concatenate([
            jnp.concatenate([wf[:D], ws[:D]], axis=1),
            jnp.concatenate([wf[D:2 * D], ws[D:2 * D]], axis=1),
        ], axis=1)
        we = jnp.concatenate([wf[2 * D:], ws[2 * D:]], axis=1)
        return wn, we

    wn1, we1 = pack(Wf1, Ws1)
    wn2, we2 = pack(Wf2, Ws2)
    b1 = jnp.concatenate([bf1, bs1])[None, :]
    b2 = jnp.concatenate([bf2, bs2])[None, :]
    wg = jnp.concatenate([Wmu, Wls], axis=1)
    src_off = jnp.concatenate([src_i, src_i + n])

    degp = _sc_deg(dst_i, n)

    h1 = _cg_layer(x, src_i, dst_i, edge_attr, wn1, we1, b1)
    h2 = _cg_layer(h1, src_i, dst_i, edge_attr, wn2, we2, b2)

    g = _mm(h2, wg)
    u2 = _u_kernel(g, degp)
    acc2 = _sc_gcn(u2, src_off, dst_i, n)
    mu, ls = _final(acc2, u2, degp, bmu[None, :], bls[None, :])
    return (mu, ls)
